# Initial kernel scaffold; baseline (speedup 1.0000x reference)
import functools

import jax
import jax.numpy as jnp
from jax import lax
from jax.experimental import pallas as pl
from jax.experimental.pallas import tpu as pltpu
from jax.experimental.pallas import tpu_sc as plsc

N = 10000
D = 128
E = 320000
NC = 2
NS = 16
NW = NC * NS
C = 128
CPW = -(-E // (C * NW))
EPAD = CPW * C * NW
ROWS_PER_TILE = N // NS
DRAIN = 125


def _spmm_body(src_hbm, dst_hbm, val_hbm, x_hbm, parts_hbm,
               acc_sh, src_v, dst_v, val_v, rows_v, zbuf, sem):
    c = lax.axis_index("c")
    s = lax.axis_index("s")
    wid = s * NC + c

    @pl.loop(0, DRAIN)
    def _zero_rows(i):
        for j in range(D // 16):
            zbuf[i, pl.ds(j * 16, 16)] = jnp.zeros((16,), jnp.float32)

    for k in range(ROWS_PER_TILE // DRAIN):
        r0 = s * ROWS_PER_TILE + k * DRAIN
        pltpu.sync_copy(zbuf, acc_sh.at[pl.ds(r0, DRAIN)])

    plsc.subcore_barrier()

    base = wid * CPW

    @pl.loop(0, CPW)
    def _edge_chunk(jc):
        row = base + jc
        pltpu.sync_copy(src_hbm.at[row], src_v)
        pltpu.sync_copy(val_hbm.at[row], val_v)
        pltpu.sync_copy(dst_hbm.at[row], dst_v)
        pltpu.async_copy(x_hbm.at[src_v], rows_v, sem).wait()

        @pl.loop(0, C)
        def _scale(e):
            v = val_v[e]
            for j in range(D // 16):
                sl = pl.ds(j * 16, 16)
                rows_v[e, sl] = rows_v[e, sl] * v

        pltpu.sync_copy(rows_v, acc_sh.at[dst_v], add=True)

    plsc.subcore_barrier()

    for k in range(ROWS_PER_TILE // DRAIN):
        r0 = s * ROWS_PER_TILE + k * DRAIN
        pltpu.sync_copy(acc_sh.at[pl.ds(r0, DRAIN)], zbuf)
        pltpu.sync_copy(zbuf, parts_hbm.at[c, pl.ds(r0, DRAIN)])


_spmm = pl.kernel(
    _spmm_body,
    out_type=jax.ShapeDtypeStruct((NC, N, D), jnp.float32),
    mesh=plsc.VectorSubcoreMesh(core_axis_name="c", subcore_axis_name="s",
                                num_cores=NC, num_subcores=NS),
    scratch_types=[
        pltpu.VMEM_SHARED((N, D), jnp.float32),
        pltpu.VMEM((C,), jnp.int32),
        pltpu.VMEM((C,), jnp.int32),
        pltpu.VMEM((C,), jnp.float32),
        pltpu.VMEM((C, D), jnp.float32),
        pltpu.VMEM((DRAIN, D), jnp.float32),
        pltpu.SemaphoreType.DMA,
    ],
)


def _combine_body(p_ref, o_ref):
    h = p_ref[0] + p_ref[1]
    n = jnp.sqrt(jnp.sum(h * h, axis=-1, keepdims=True))
    o_ref[...] = h / jnp.maximum(n, 1e-12)


_BR = 1000


def _combine(parts):
    return pl.pallas_call(
        _combine_body,
        grid=(N // _BR,),
        in_specs=[pl.BlockSpec((NC, _BR, D), lambda i: (0, i, 0))],
        out_specs=pl.BlockSpec((_BR, D), lambda i: (i, 0)),
        out_shape=jax.ShapeDtypeStruct((N, D), jnp.float32),
    )(parts)


@jax.jit
def kernel(x, adj_indices, adj_values):
    pad = EPAD - E
    dst = jnp.pad(adj_indices[0], (0, pad)).reshape(EPAD // C, C)
    src = jnp.pad(adj_indices[1], (0, pad)).reshape(EPAD // C, C)
    val = jnp.pad(adj_values, (0, pad)).reshape(EPAD // C, C)
    parts = _spmm(src, dst, val, x)
    return _combine(parts)


# speedup vs baseline: 3.7330x; 3.7330x over previous
import jax
import jax.numpy as jnp
from jax import lax
from jax.experimental import pallas as pl
from jax.experimental.pallas import tpu as pltpu
from jax.experimental.pallas import tpu_sc as plsc

N = 10000
D = 128
E = 320000
NC = 2
NS = 16
NW = NC * NS
C = 128
CPW = -(-E // (C * NW))
EPAD = CPW * C * NW
NPAD = 10240
RPT = NPAD // NS
DRAIN = 128


def _spmm_body(src_hbm, dst_hbm, val_hbm, x_hbm, parts_hbm,
               acc_sh, src_v, dst_v, val_v, rows_v, zbuf, sem):
    c = lax.axis_index("c")
    s = lax.axis_index("s")
    wid = s * NC + c

    @pl.loop(0, DRAIN)
    def _zero_rows(i):
        for j in range(D // 16):
            zbuf[i, pl.ds(j * 16, 16)] = jnp.zeros((16,), jnp.float32)

    for k in range(RPT // DRAIN):
        r0 = s * RPT + k * DRAIN
        pltpu.sync_copy(zbuf, acc_sh.at[pl.ds(r0, DRAIN)])

    plsc.subcore_barrier()

    base = wid * CPW * C

    @pl.loop(0, CPW)
    def _edge_chunk(jc):
        e0 = base + jc * C
        pltpu.sync_copy(src_hbm.at[pl.ds(e0, C)], src_v)
        pltpu.sync_copy(val_hbm.at[pl.ds(e0, C)], val_v)
        pltpu.sync_copy(dst_hbm.at[pl.ds(e0, C)], dst_v)
        pltpu.async_copy(x_hbm.at[src_v], rows_v, sem).wait()

        @pl.loop(0, C // 16)
        def _scale(eb):
            vv = val_v[pl.ds(eb * 16, 16)]
            for l in range(16):
                v = vv[l]
                e = eb * 16 + l
                for j in range(D // 16):
                    sl = pl.ds(j * 16, 16)
                    rows_v[e, sl] = rows_v[e, sl] * v

        pltpu.sync_copy(rows_v, acc_sh.at[dst_v], add=True)

    plsc.subcore_barrier()

    for k in range(RPT // DRAIN):
        r0 = s * RPT + k * DRAIN
        pltpu.sync_copy(acc_sh.at[pl.ds(r0, DRAIN)], zbuf)
        pltpu.sync_copy(zbuf, parts_hbm.at[c, pl.ds(r0, DRAIN)])


_spmm = pl.kernel(
    _spmm_body,
    out_type=jax.ShapeDtypeStruct((NC, NPAD, D), jnp.float32),
    mesh=plsc.VectorSubcoreMesh(core_axis_name="c", subcore_axis_name="s",
                                num_cores=NC, num_subcores=NS),
    scratch_types=[
        pltpu.VMEM_SHARED((NPAD, D), jnp.float32),
        pltpu.VMEM((C,), jnp.int32),
        pltpu.VMEM((C,), jnp.int32),
        pltpu.VMEM((C,), jnp.float32),
        pltpu.VMEM((C, D), jnp.float32),
        pltpu.VMEM((DRAIN, D), jnp.float32),
        pltpu.SemaphoreType.DMA,
    ],
)


def _combine_body(p_ref, o_ref):
    h = p_ref[0] + p_ref[1]
    n = jnp.sqrt(jnp.sum(h * h, axis=-1, keepdims=True))
    o_ref[...] = h / jnp.maximum(n, 1e-12)


_BR = 1000


def _combine(parts):
    return pl.pallas_call(
        _combine_body,
        grid=(N // _BR,),
        in_specs=[pl.BlockSpec((NC, _BR, D), lambda i: (0, i, 0))],
        out_specs=pl.BlockSpec((_BR, D), lambda i: (i, 0)),
        out_shape=jax.ShapeDtypeStruct((N, D), jnp.float32),
    )(parts)


@jax.jit
def kernel(x, adj_indices, adj_values):
    pad = EPAD - E
    dst = jnp.pad(adj_indices[0], (0, pad))
    src = jnp.pad(adj_indices[1], (0, pad))
    val = jnp.pad(adj_values, (0, pad))
    parts = _spmm(src, dst, val, x)
    return _combine(parts[:, :N])
